# baseline (speedup 1.0000x reference)
import functools

import jax
import jax.numpy as jnp
from jax import lax
from jax.experimental import pallas as pl
from jax.experimental.pallas import tpu as pltpu
from jax.experimental.pallas import tpu_sc as plsc

_K = 16
_KB = 256
_SG = 128
_NW = 32
_CH = 64
_CAP = 320


def _dist_tau_kernel(x_ref, y_ref, dist_ref, tau_ref):
    sg = pl.program_id(2)
    x = x_ref[0, :, pl.ds(sg * _SG, _SG)]
    y = y_ref[0]
    xn = x / jnp.maximum(jnp.sqrt(jnp.sum(x * x, axis=0, keepdims=True)), 1e-12)
    yn = y / jnp.maximum(jnp.sqrt(jnp.sum(y * y, axis=0, keepdims=True)), 1e-12)
    x2 = jnp.sum(xn * xn, axis=0)
    y2 = jnp.sum(yn * yn, axis=0)
    inner = jax.lax.dot_general(
        yn.astype(jnp.bfloat16), xn.astype(jnp.bfloat16),
        (((0,), (0,)), ((), ())),
        preferred_element_type=jnp.float32,
    )
    dist = (x2[None, :] + (-2.0) * inner) + y2[:, None]
    dist_ref[0, 0] = dist
    tau_ref[0, 0, 0, 0] = jnp.min(dist, axis=0)


def _sc_topk_kernel(dist_hbm, taup_hbm, out_hbm, chunk0, chunk1, taubuf, cval,
                    cidx, cntbuf, outbuf, sem0, sem1):
    nb, nsg_b, n, _ = dist_hbm.shape
    q_w = 256
    nsg = q_w // _SG
    nvs = _SG // 16
    nch = n // _CH
    nw_b = n // q_w
    wid = lax.axis_index("s") * 2 + lax.axis_index("c")
    b = wid // nw_b
    sg0 = (wid % nw_b) * nsg
    lane = lax.iota(jnp.int32, 16)
    inf = jnp.float32(jnp.inf)
    bufs = (chunk0, chunk1)
    sems = (sem0, sem1)

    def start_fetch(sg, kc, buf, sem):
        pltpu.async_copy(
            dist_hbm.at[b, sg0 + sg, pl.ds(kc * _CH, _CH), :], buf, sem,
        )

    def wait_fetch(buf, sem):
        pltpu.make_async_copy(dist_hbm.at[b, 0, pl.ds(0, _CH), :],
                              buf, sem).wait()

    start_fetch(0, 0, bufs[0], sems[0])

    def sg_body(sg, _):
        pltpu.sync_copy(taup_hbm.at[b, :, sg0 + sg, 0, :], taubuf)
        taus = []
        for s in range(nvs):
            tv = taubuf[0, pl.ds(s * 16, 16)]
            for kg in range(1, 16):
                tv = jnp.maximum(tv, taubuf[kg, pl.ds(s * 16, 16)])
            taus.append(tv)
        rows = [lane + s * 16 for s in range(nvs)]

        def filter_chunk(buf, carry):
            def col_body(j, carry2):
                *cs, jv2 = carry2
                cs = list(cs)
                for s in range(nvs):
                    v = buf[j, pl.ds(s * 16, 16)]
                    m = v <= taus[s]
                    pos = jnp.minimum(cs[s], _CAP - 1)
                    plsc.store_scatter(cval, [rows[s], pos], v, mask=m)
                    plsc.store_scatter(cidx, [rows[s], pos], jv2, mask=m)
                    cs[s] = cs[s] + m.astype(jnp.int32)
                return (*cs, jv2 + 1)

            return plsc.parallel_loop(
                0, _CH, step=1, unroll=8, carry=carry
            )(col_body)

        def pair_body(p, carry):
            wait_fetch(bufs[0], sems[0])
            start_fetch(sg, 2 * p + 1, bufs[1], sems[1])
            carry = filter_chunk(bufs[0], carry)
            wait_fetch(bufs[1], sems[1])

            @pl.when(p < nch // 2 - 1)
            def _():
                start_fetch(sg, 2 * p + 2, bufs[0], sems[0])

            @pl.when((p == nch // 2 - 1) & (sg < nsg - 1))
            def _():
                start_fetch(sg + 1, 0, bufs[0], sems[0])

            return filter_chunk(bufs[1], carry)

        carry0 = (*([jnp.zeros((16,), jnp.int32)] * nvs),
                  jnp.zeros((16,), jnp.int32))
        *cnts, jv = lax.fori_loop(0, nch // 2, pair_body, carry0)
        cnts = list(cnts)

        for s in range(nvs):
            cntbuf[pl.ds(s * 16, 16)] = jnp.minimum(cnts[s], _CAP)

        def row_body(rr, _):
            ncv = plsc.load_gather(cntbuf, [jnp.full((16,), rr, jnp.int32)])
            n_c = jnp.max(ncv)

            def merge_body(t, carry):
                rk, ri = carry
                valid = (lane + t * 16) < ncv
                vals = cval[rr, pl.ds(t * 16, 16)]
                idxv = cidx[rr, pl.ds(t * 16, 16)]
                ck, ci = plsc.sort_key_val(
                    jnp.where(valid, vals, inf),
                    jnp.where(valid, idxv, jnp.int32(2**30)),
                )
                bk = lax.rev(rk, (0,))
                bi = lax.rev(ri, (0,))
                take_a = (ck < bk) | ((ck == bk) & (ci < bi))
                return tuple(plsc.sort_key_val(
                    jnp.where(take_a, ck, bk), jnp.where(take_a, ci, bi)
                ))

            rk0 = jnp.full((16,), inf)
            ri0 = jnp.full((16,), 2**30, dtype=jnp.int32)
            _, ri = lax.fori_loop(0, (n_c + 15) // 16, merge_body, (rk0, ri0))
            outbuf[sg * _SG + rr] = ri
            return jnp.int32(0)

        plsc.parallel_loop(0, _SG, step=1, unroll=2,
                           carry=jnp.int32(0))(row_body)
        return 0

    lax.fori_loop(0, nsg, sg_body, 0)
    pltpu.sync_copy(outbuf, out_hbm.at[pl.ds(wid * q_w, q_w)])


def kernel(x, y):
    b, c, n, _ = x.shape
    xs = x[..., 0]
    ys = y[..., 0]
    nsg_b = n // _SG
    dist_g, tau_p = pl.pallas_call(
        _dist_tau_kernel,
        grid=(b, n // _KB, nsg_b),
        in_specs=[
            pl.BlockSpec((1, c, n), lambda bi, i, g: (bi, 0, 0)),
            pl.BlockSpec((1, c, _KB), lambda bi, i, g: (bi, 0, i)),
        ],
        out_specs=[
            pl.BlockSpec((1, 1, _KB, _SG), lambda bi, i, g: (bi, g, i, 0)),
            pl.BlockSpec((1, 1, 1, 1, _SG), lambda bi, i, g: (bi, i, g, 0, 0)),
        ],
        out_shape=[
            jax.ShapeDtypeStruct((b, nsg_b, n, _SG), jnp.float32),
            jax.ShapeDtypeStruct((b, n // _KB, nsg_b, 1, _SG), jnp.float32),
        ],
    )(xs, ys)

    rows = b * n
    sc_topk = functools.partial(
        pl.kernel,
        out_type=jax.ShapeDtypeStruct((rows, _K), jnp.int32),
        mesh=plsc.VectorSubcoreMesh(core_axis_name="c", subcore_axis_name="s"),
        compiler_params=pltpu.CompilerParams(
            needs_layout_passes=False, use_tc_tiling_on_sc=False
        ),
        scratch_types=[
            pltpu.VMEM((_CH, _SG), jnp.float32),
            pltpu.VMEM((_CH, _SG), jnp.float32),
            pltpu.VMEM((16, _SG), jnp.float32),
            pltpu.VMEM((_SG, _CAP), jnp.float32),
            pltpu.VMEM((_SG, _CAP), jnp.int32),
            pltpu.VMEM((_SG,), jnp.int32),
            pltpu.VMEM((256, _K), jnp.int32),
            pltpu.SemaphoreType.DMA,
            pltpu.SemaphoreType.DMA,
        ],
    )(_sc_topk_kernel)
    nn_idx = sc_topk(dist_g, tau_p).reshape(b, n, _K)

    center_idx = jnp.broadcast_to(
        jnp.arange(n, dtype=jnp.int32)[None, :, None], (b, n, _K)
    )
    return jnp.stack((nn_idx, center_idx), axis=0)


# speedup vs baseline: 2.0338x; 2.0338x over previous
import functools

import jax
import jax.numpy as jnp
from jax import lax
from jax.experimental import pallas as pl
from jax.experimental.pallas import tpu as pltpu
from jax.experimental.pallas import tpu_sc as plsc

_K = 16
_KB = 256
_NW = 32
_SG = 64
_CH = 256
_CAP = 448


def _dist_tau_kernel(x_ref, y_ref, dist_ref, tau_ref):
    x = x_ref[0]
    y = y_ref[0]
    xn = x / jnp.maximum(jnp.sqrt(jnp.sum(x * x, axis=0, keepdims=True)), 1e-12)
    yn = y / jnp.maximum(jnp.sqrt(jnp.sum(y * y, axis=0, keepdims=True)), 1e-12)
    x2 = jnp.sum(xn * xn, axis=0)
    y2 = jnp.sum(yn * yn, axis=0)
    inner = jax.lax.dot_general(
        yn.astype(jnp.bfloat16), xn.astype(jnp.bfloat16),
        (((0,), (0,)), ((), ())),
        preferred_element_type=jnp.float32,
    )
    dist = (x2[None, :] + (-2.0) * inner) + y2[:, None]
    dist_ref[0] = dist
    tau_ref[0, 0, 0] = jnp.min(dist, axis=0)


def _sc_topk_kernel(dist_hbm, taup_hbm, out_hbm, chunk0, chunk1, taubuf, cval,
                    cidx, cntbuf, outbuf, sem0, sem1):
    nb, n, _ = dist_hbm.shape
    q_w = 256
    nsg = q_w // _SG
    nvs = _SG // 16
    nch = n // _CH
    nw_b = n // q_w
    wid = lax.axis_index("s") * 2 + lax.axis_index("c")
    b = wid // nw_b
    qbase = (wid % nw_b) * q_w
    pltpu.sync_copy(taup_hbm.at[b, :, 0, pl.ds(qbase, q_w)], taubuf)
    lane = lax.iota(jnp.int32, 16)
    inf = jnp.float32(jnp.inf)
    bufs = (chunk0, chunk1)
    sems = (sem0, sem1)

    def start_fetch(sg, kc, buf, sem):
        pltpu.async_copy(
            dist_hbm.at[b, pl.ds(kc * _CH, _CH), pl.ds(qbase + sg * _SG, _SG)],
            buf, sem,
        )

    def wait_fetch(buf, sem):
        pltpu.make_async_copy(dist_hbm.at[b, pl.ds(0, _CH), pl.ds(0, _SG)],
                              buf, sem).wait()

    start_fetch(0, 0, bufs[0], sems[0])

    def sg_body(sg, _):
        taus = []
        for s in range(nvs):
            tv = taubuf[0, pl.ds(sg * _SG + s * 16, 16)]
            for kg in range(1, 16):
                tv = jnp.maximum(tv, taubuf[kg, pl.ds(sg * _SG + s * 16, 16)])
            taus.append(tv)

        cnts = [jnp.zeros((16,), jnp.int32)] * nvs
        jv = jnp.zeros((16,), jnp.int32)
        for kc in range(nch):
            wait_fetch(bufs[kc % 2], sems[kc % 2])
            if kc < nch - 1:
                start_fetch(sg, kc + 1, bufs[(kc + 1) % 2], sems[(kc + 1) % 2])
            else:
                @pl.when(sg < nsg - 1)
                def _():
                    start_fetch(sg + 1, 0, bufs[0], sems[0])

            def col_body(j, carry, kc=kc):
                *cs, jv2 = carry
                cs = list(cs)
                for s in range(nvs):
                    v = bufs[kc % 2][j, pl.ds(s * 16, 16)]
                    m = v <= taus[s]
                    pos = jnp.minimum(cs[s], _CAP - 1)
                    row = lane + s * 16
                    plsc.store_scatter(cval, [row, pos], v, mask=m)
                    plsc.store_scatter(cidx, [row, pos], jv2, mask=m)
                    cs[s] = cs[s] + m.astype(jnp.int32)
                return (*cs, jv2 + 1)

            *cnts, jv = plsc.parallel_loop(
                0, _CH, step=1, unroll=8, carry=(*cnts, jv)
            )(col_body)
            cnts = list(cnts)

        for s in range(nvs):
            cntbuf[pl.ds(s * 16, 16)] = jnp.minimum(cnts[s], _CAP)

        def row_body(rr, _):
            ncv = plsc.load_gather(cntbuf, [jnp.full((16,), rr, jnp.int32)])
            n_c = jnp.max(ncv)

            def merge_body(t, carry):
                rk, ri = carry
                valid = (lane + t * 16) < ncv
                vals = cval[rr, pl.ds(t * 16, 16)]
                idxv = cidx[rr, pl.ds(t * 16, 16)]
                ck, ci = plsc.sort_key_val(
                    jnp.where(valid, vals, inf),
                    jnp.where(valid, idxv, jnp.int32(2**30)),
                )
                bk = lax.rev(rk, (0,))
                bi = lax.rev(ri, (0,))
                take_a = (ck < bk) | ((ck == bk) & (ci < bi))
                return tuple(plsc.sort_key_val(
                    jnp.where(take_a, ck, bk), jnp.where(take_a, ci, bi)
                ))

            rk0 = jnp.full((16,), inf)
            ri0 = jnp.full((16,), 2**30, dtype=jnp.int32)
            _, ri = lax.fori_loop(0, (n_c + 15) // 16, merge_body, (rk0, ri0))
            outbuf[sg * _SG + rr] = ri
            return jnp.int32(0)

        plsc.parallel_loop(0, _SG, step=1, unroll=2,
                           carry=jnp.int32(0))(row_body)
        return 0

    lax.fori_loop(0, nsg, sg_body, 0)
    pltpu.sync_copy(outbuf, out_hbm.at[pl.ds(wid * q_w, q_w)])


def kernel(x, y):
    b, c, n, _ = x.shape
    xs = x[..., 0]
    ys = y[..., 0]
    dist_t, tau_p = pl.pallas_call(
        _dist_tau_kernel,
        grid=(b, n // _KB),
        in_specs=[
            pl.BlockSpec((1, c, n), lambda bi, i: (bi, 0, 0)),
            pl.BlockSpec((1, c, _KB), lambda bi, i: (bi, 0, i)),
        ],
        out_specs=[
            pl.BlockSpec((1, _KB, n), lambda bi, i: (bi, i, 0)),
            pl.BlockSpec((1, 1, 1, n), lambda bi, i: (bi, i, 0, 0)),
        ],
        out_shape=[
            jax.ShapeDtypeStruct((b, n, n), jnp.float32),
            jax.ShapeDtypeStruct((b, n // _KB, 1, n), jnp.float32),
        ],
    )(xs, ys)

    rows = b * n
    sc_topk = functools.partial(
        pl.kernel,
        out_type=jax.ShapeDtypeStruct((rows, _K), jnp.int32),
        mesh=plsc.VectorSubcoreMesh(core_axis_name="c", subcore_axis_name="s"),
        compiler_params=pltpu.CompilerParams(
            needs_layout_passes=False, use_tc_tiling_on_sc=False
        ),
        scratch_types=[
            pltpu.VMEM((_CH, _SG), jnp.float32),
            pltpu.VMEM((_CH, _SG), jnp.float32),
            pltpu.VMEM((n // _KB, rows // _NW), jnp.float32),
            pltpu.VMEM((_SG, _CAP), jnp.float32),
            pltpu.VMEM((_SG, _CAP), jnp.int32),
            pltpu.VMEM((_SG,), jnp.int32),
            pltpu.VMEM((rows // _NW, _K), jnp.int32),
            pltpu.SemaphoreType.DMA,
            pltpu.SemaphoreType.DMA,
        ],
    )(_sc_topk_kernel)
    nn_idx = sc_topk(dist_t, tau_p).reshape(b, n, _K)

    center_idx = jnp.broadcast_to(
        jnp.arange(n, dtype=jnp.int32)[None, :, None], (b, n, _K)
    )
    return jnp.stack((nn_idx, center_idx), axis=0)
